# baseline (speedup 1.0000x reference)
import jax
import jax.numpy as jnp
from jax import lax
from jax.experimental import pallas as pl
from jax.experimental.pallas import tpu as pltpu
from jax.experimental.pallas import tpu_sc as plsc

N0 = 10000
NV = 64
RV = 160
NP = NV * RV
D = 256
E = 160000
EP = E + 2304
C = 128
CHJ = 128
NJ = D // CHJ
RACC = 168
F32 = jnp.float32

_mesh = plsc.VectorSubcoreMesh(core_axis_name="c", subcore_axis_name="s")


SCE = 2048
NCSC = SCE // C


def _edge_body(ds_hbm, sr_hbm, bounds_hbm, pstack, qstack,
               s_out, mn_out, mx_out, deg_out,
               bbuf, dsb, srb, idx_a, row_a, idx_b, row_b, qbuf_a, qbuf_b,
               pbuf, acc_s, acc_mn, acc_mx, degacc, sem_a, sem_b):
    wid = lax.axis_index("s") * 2 + lax.axis_index("c")
    lanes = jnp.arange(16, dtype=jnp.int32)
    ones16 = jnp.full((16,), 1.0, dtype=F32)
    zeros16 = jnp.zeros((16,), F32)
    inf16 = jnp.full((16,), jnp.inf, F32)
    ninf16 = jnp.full((16,), -jnp.inf, F32)

    pltpu.sync_copy(bounds_hbm, bbuf)
    bch = bbuf[pl.ds(2 * wid, 16)]
    s0, s1, s2 = bch[0], bch[1], bch[2]

    def step_body(step, _):
        sub = step >> 1
        j = step & 1
        v = 2 * wid + sub
        lo = v * RV
        rowidx = j * NV + v
        qbase = j * NP
        start = jnp.where(sub == 0, s0, s1)
        end = jnp.where(sub == 0, s1, s2)
        start0 = (start // C) * C
        trips = lax.div(end - start0 + (C - 1), C)
        nsc = lax.div(trips + (NCSC - 1), NCSC)

        def initr(r, _):
            for vv in range(CHJ // 16):
                sl = pl.ds(r * CHJ + vv * 16, 16)
                acc_s[sl] = zeros16
                acc_mn[sl] = inf16
                acc_mx[sl] = ninf16
            degacc[pl.ds(r * 16, 16)] = zeros16
            return 0

        lax.fori_loop(0, RACC, initr, 0)

        pltpu.sync_copy(pstack.at[rowidx], pbuf.at[pl.ds(0, RV * CHJ)])

        def sanitize(t_local, e0, idxb, rowb):
            for vv in range(C // 16):
                sl = pl.ds(vv * 16, 16)
                sll = pl.ds(t_local * C + vv * 16, 16)
                row = dsb[sll] - lo
                eidx = e0 + (lanes + vv * 16)
                ok = (row >= 0) & (row < RV) & (eidx < end)
                rowb[sl] = jnp.where(ok, row, RV)
                idxb[sl] = jnp.where(ok, srb[sll] + qbase, qbase)

        def process(idxb, rowb, qb):
            def group_body(g, _):
                rows16 = rowb[pl.ds(g * 16, 16)]
                for lane in range(16):
                    row = rows16[lane]
                    ro = row * CHJ
                    for vv in range(CHJ // 16):
                        sl = pl.ds(ro + vv * 16, 16)
                        m = jnp.maximum(
                            pbuf[sl] + qb[g * 16 + lane,
                                          pl.ds(vv * 16, 16)], 0.0)
                        plsc.addupdate(acc_s.at[sl], m)
                        acc_mn[sl] = jnp.minimum(acc_mn[sl], m)
                        acc_mx[sl] = jnp.maximum(acc_mx[sl], m)
                    plsc.addupdate(degacc.at[pl.ds(row * 16, 16)], ones16)
                return 0

            lax.fori_loop(0, C // 16, group_body, 0)

        def super_body(sc, _):
            sc0 = start0 + sc * SCE
            pltpu.sync_copy(ds_hbm.at[pl.ds(sc0, SCE)], dsb)
            pltpu.sync_copy(sr_hbm.at[pl.ds(sc0, SCE)], srb)
            trips_sc = jnp.minimum(trips - sc * NCSC, NCSC)
            npair = lax.div(trips_sc + 1, 2)

            def clamp(t):
                return jnp.minimum(t, trips_sc - 1)

            sanitize(0, sc0, idx_a, row_a)
            pltpu.async_copy(qstack.at[idx_a], qbuf_a, sem_a)

            def pair_body(tp, _):
                t1 = 2 * tp + 1
                sanitize(clamp(t1), sc0 + t1 * C, idx_b, row_b)
                pltpu.async_copy(qstack.at[idx_b], qbuf_b, sem_b)
                pltpu.make_async_copy(qstack.at[idx_a], qbuf_a, sem_a).wait()
                process(idx_a, row_a, qbuf_a)
                t2 = 2 * tp + 2
                sanitize(clamp(t2), sc0 + t2 * C, idx_a, row_a)
                pltpu.async_copy(qstack.at[idx_a], qbuf_a, sem_a)
                pltpu.make_async_copy(qstack.at[idx_b], qbuf_b, sem_b).wait()
                process(idx_b, row_b, qbuf_b)
                return 0

            lax.fori_loop(0, npair, pair_body, 0)
            pltpu.make_async_copy(qstack.at[idx_a], qbuf_a, sem_a).wait()
            return 0

        lax.fori_loop(0, nsc, super_body, 0)

        pltpu.sync_copy(acc_s.at[pl.ds(0, RV * CHJ)], s_out.at[rowidx])
        pltpu.sync_copy(acc_mn.at[pl.ds(0, RV * CHJ)], mn_out.at[rowidx])
        pltpu.sync_copy(acc_mx.at[pl.ds(0, RV * CHJ)], mx_out.at[rowidx])
        pltpu.sync_copy(degacc, deg_out.at[v])
        return 0

    lax.fori_loop(0, 2 * NJ, step_body, 0)


_edge_kernel = pl.kernel(
    _edge_body,
    out_type=(
        jax.ShapeDtypeStruct((NJ * NV, RV * CHJ), F32),
        jax.ShapeDtypeStruct((NJ * NV, RV * CHJ), F32),
        jax.ShapeDtypeStruct((NJ * NV, RV * CHJ), F32),
        jax.ShapeDtypeStruct((NV, RACC * 16), F32),
    ),
    mesh=_mesh,
    scratch_types=[
        pltpu.VMEM((80,), jnp.int32),
        pltpu.VMEM((SCE,), jnp.int32),
        pltpu.VMEM((SCE,), jnp.int32),
        pltpu.VMEM((C,), jnp.int32),
        pltpu.VMEM((C,), jnp.int32),
        pltpu.VMEM((C,), jnp.int32),
        pltpu.VMEM((C,), jnp.int32),
        pltpu.VMEM((C, CHJ), F32),
        pltpu.VMEM((C, CHJ), F32),
        pltpu.VMEM((RACC * CHJ,), F32),
        pltpu.VMEM((RACC * CHJ,), F32),
        pltpu.VMEM((RACC * CHJ,), F32),
        pltpu.VMEM((RACC * CHJ,), F32),
        pltpu.VMEM((RACC * 16,), F32),
        pltpu.SemaphoreType.DMA,
        pltpu.SemaphoreType.DMA,
    ],
)


BF16 = jnp.bfloat16


def _dotb(u, w):
    return jnp.dot(u.astype(BF16), w.astype(BF16),
                   preferred_element_type=F32)


def _pre_body(x_ref, wd_ref, ws_ref, b_ref, p_ref, q_ref):
    xb = x_ref[...]
    p_ref[...] = _dotb(xb, wd_ref[...]) + b_ref[...]
    q_ref[...] = _dotb(xb, ws_ref[...])


def _post_body(x_ref, s_ref, mn_ref, mx_ref, deg_ref,
               a_ref, b_ref, c_ref, d_ref, e_ref, f_ref, g_ref,
               pb_ref, idl_ref, lw_ref, lb_ref, o_ref):
    h = x_ref[...]
    deg = deg_ref[...]
    he = deg > 0
    mean = s_ref[...] / jnp.maximum(deg, 1.0)
    mn = jnp.where(he, mn_ref[...], 0.0)
    mx = jnp.where(he, mx_ref[...], 0.0)
    amp = jnp.log(1.0 + deg) * idl_ref[...]
    base = _dotb(h, a_ref[...]) + _dotb(mean, b_ref[...]) \
        + _dotb(mn, c_ref[...]) + _dotb(mx, d_ref[...]) + pb_ref[...]
    scaled = _dotb(mean, e_ref[...]) + _dotb(mn, f_ref[...]) \
        + _dotb(mx, g_ref[...])
    y = base + amp * scaled
    o_ref[...] = jnp.maximum(_dotb(y, lw_ref[...]) + lb_ref[...], 0.0)


def _fc_body(x_ref, w_ref, b_ref, o_ref):
    o_ref[...] = _dotb(x_ref[...], w_ref[...]) + b_ref[...]


def _row_spec(rb):
    return pl.BlockSpec((rb, D), lambda i: (i, 0))


def _full_spec(shape):
    return pl.BlockSpec(shape, lambda i: tuple(0 for _ in shape))


_BR = 2048


def _pre_call(xp, wd, ws, bias):
    return pl.pallas_call(
        _pre_body,
        grid=(NP // _BR,),
        in_specs=[_row_spec(_BR), _full_spec((D, D)), _full_spec((D, D)),
                  _full_spec((1, D))],
        out_specs=[_row_spec(_BR), _row_spec(_BR)],
        out_shape=[jax.ShapeDtypeStruct((NP, D), F32)] * 2,
    )(xp, wd, ws, bias)


def _post_call(xp, s, mn, mx, deg, mats, pb, idl, lw, lb):
    return pl.pallas_call(
        _post_body,
        grid=(NP // _BR,),
        in_specs=[_row_spec(_BR)] * 4
        + [pl.BlockSpec((_BR, 1), lambda i: (i, 0))]
        + [_full_spec((D, D))] * 7
        + [_full_spec((1, D)), _full_spec((1, 1)), _full_spec((D, D)),
           _full_spec((1, D))],
        out_specs=_row_spec(_BR),
        out_shape=jax.ShapeDtypeStruct((NP, D), F32),
    )(xp, s, mn, mx, deg, *mats, pb, idl, lw, lb)


def _fc_call(x, w, b):
    return pl.pallas_call(
        _fc_body,
        grid=(5,),
        in_specs=[pl.BlockSpec((2000, D), lambda i: (i, 0)),
                  _full_spec((D, D)), _full_spec((1, D))],
        out_specs=pl.BlockSpec((2000, D), lambda i: (i, 0)),
        out_shape=jax.ShapeDtypeStruct((N0, D), F32),
    )(x, w, b)


def _blockdiag(w0, w1):
    k = w0.shape[0]
    out = jnp.zeros((2 * k, 2 * w0.shape[1]), F32)
    out = out.at[:k, :w0.shape[1]].set(w0)
    out = out.at[k:, w0.shape[1]:].set(w1)
    return out


def kernel(x, edge_index, pre_W, pre_b, post_W, post_b, lin_W, lin_b, delta, fc_W, fc_b):
    src = edge_index[0]
    dst = edge_index[1]
    tin = D // 2

    xp = jnp.pad(x, ((0, NP - N0), (0, 0)))

    perm = jnp.argsort(dst)
    ds_s = dst[perm]
    sr_s = src[perm]
    bounds = jnp.searchsorted(
        ds_s, jnp.arange(NV + 1, dtype=jnp.int32) * RV).astype(jnp.int32)
    bounds = jnp.pad(bounds, (0, 80 - (NV + 1)), constant_values=E)
    ds_p = jnp.concatenate(
        [ds_s, jnp.full((EP - E,), 1 << 22, jnp.int32)])
    sr_p = jnp.concatenate([sr_s, jnp.zeros((EP - E,), jnp.int32)])

    num_layers = pre_W.shape[0]
    for l in range(num_layers):
        wd = _blockdiag(pre_W[l, 0][:tin], pre_W[l, 1][:tin])
        ws = _blockdiag(pre_W[l, 0][tin:], pre_W[l, 1][tin:])
        bias = jnp.concatenate([pre_b[l, 0], pre_b[l, 1]]).reshape(1, D)

        p2, q2 = _pre_call(xp, wd, ws, bias)
        pstack = p2.reshape(NV, RV, NJ, CHJ).transpose(2, 0, 1, 3) \
                   .reshape(NJ * NV, RV * CHJ)
        qstack = jnp.concatenate(
            [q2[:, j * CHJ:(j + 1) * CHJ] for j in range(NJ)], axis=0)

        s_f, mn_f, mx_f, deg_f = _edge_kernel(
            ds_p, sr_p, bounds, pstack, qstack)
        s = s_f.reshape(NJ, NV, RV, CHJ).transpose(1, 2, 0, 3).reshape(NP, D)
        mn = mn_f.reshape(NJ, NV, RV, CHJ).transpose(1, 2, 0, 3).reshape(NP, D)
        mx = mx_f.reshape(NJ, NV, RV, CHJ).transpose(1, 2, 0, 3).reshape(NP, D)
        deg = deg_f.reshape(NV, RACC, 16)[:, :RV, 0].reshape(NP, 1)

        mats = []
        for r0 in range(0, 7 * tin, tin):
            mats.append(_blockdiag(post_W[l, 0][r0:r0 + tin],
                                   post_W[l, 1][r0:r0 + tin]))
        pb = jnp.concatenate([post_b[l, 0], post_b[l, 1]]).reshape(1, D)
        idl = (1.0 / delta[l]).reshape(1, 1)

        xp = _post_call(xp, s, mn, mx, deg, mats, pb, idl,
                        lin_W[l], lin_b[l].reshape(1, D))

    return _fc_call(xp[:N0], fc_W, fc_b.reshape(1, D))


# speedup vs baseline: 1.4056x; 1.4056x over previous
import jax
import jax.numpy as jnp
from jax import lax
from jax.experimental import pallas as pl
from jax.experimental.pallas import tpu as pltpu
from jax.experimental.pallas import tpu_sc as plsc

N0 = 10000
NV = 64
RV = 160
NP = NV * RV
D = 256
E = 160000
EP = E + 128
C = 128
CHJ = 128
NJ = D // CHJ
RACC = 168
F32 = jnp.float32

_mesh = plsc.VectorSubcoreMesh(core_axis_name="c", subcore_axis_name="s")


def _edge_body(ds_hbm, sr_hbm, bounds_hbm, pstack, qstack,
               s_out, mn_out, mx_out, deg_out,
               bbuf, dbuf, sbuf, idxbuf, rowbuf, qbuf, pbuf,
               acc_s, acc_mn, acc_mx, degacc, sem):
    wid = lax.axis_index("s") * 2 + lax.axis_index("c")
    lanes = jnp.arange(16, dtype=jnp.int32)
    ones16 = jnp.full((16,), 1.0, dtype=F32)
    zeros16 = jnp.zeros((16,), F32)
    inf16 = jnp.full((16,), jnp.inf, F32)
    ninf16 = jnp.full((16,), -jnp.inf, F32)

    pltpu.sync_copy(bounds_hbm, bbuf)
    bch = bbuf[pl.ds(2 * wid, 16)]
    s0, s1, s2 = bch[0], bch[1], bch[2]

    def step_body(step, _):
        sub = step >> 1
        j = step & 1
        v = 2 * wid + sub
        lo = v * RV
        rowidx = j * NV + v
        qbase = j * NP
        start = jnp.where(sub == 0, s0, s1)
        end = jnp.where(sub == 0, s1, s2)
        start0 = (start // C) * C
        trips = lax.div(end - start0 + (C - 1), C)

        def initr(r, _):
            for vv in range(CHJ // 16):
                sl = pl.ds(r * CHJ + vv * 16, 16)
                acc_s[sl] = zeros16
                acc_mn[sl] = inf16
                acc_mx[sl] = ninf16
            degacc[pl.ds(r * 16, 16)] = zeros16
            return 0

        lax.fori_loop(0, RACC, initr, 0)

        pltpu.sync_copy(pstack.at[rowidx], pbuf.at[pl.ds(0, RV * CHJ)])

        def chunk_body(t, _):
            e0 = start0 + t * C
            pltpu.sync_copy(ds_hbm.at[pl.ds(e0, C)], dbuf)
            pltpu.sync_copy(sr_hbm.at[pl.ds(e0, C)], sbuf)
            for vv in range(C // 16):
                sl = pl.ds(vv * 16, 16)
                row = dbuf[sl] - lo
                ok = (row >= 0) & (row < RV)
                rowbuf[sl] = jnp.where(ok, row, RV)
                idxbuf[sl] = jnp.where(ok, sbuf[sl] + qbase, qbase)
            pltpu.async_copy(qstack.at[idxbuf], qbuf, sem).wait()

            def group_body(g, _):
                rows16 = rowbuf[pl.ds(g * 16, 16)]
                for lane in range(16):
                    row = rows16[lane]
                    ro = row * CHJ
                    for vv in range(CHJ // 16):
                        sl = pl.ds(ro + vv * 16, 16)
                        m = jnp.maximum(
                            pbuf[sl] + qbuf[g * 16 + lane,
                                            pl.ds(vv * 16, 16)], 0.0)
                        plsc.addupdate(acc_s.at[sl], m)
                        acc_mn[sl] = jnp.minimum(acc_mn[sl], m)
                        acc_mx[sl] = jnp.maximum(acc_mx[sl], m)
                    plsc.addupdate(degacc.at[pl.ds(row * 16, 16)], ones16)
                return 0

            lax.fori_loop(0, C // 16, group_body, 0)
            return 0

        lax.fori_loop(0, trips, chunk_body, 0)

        pltpu.sync_copy(acc_s.at[pl.ds(0, RV * CHJ)], s_out.at[rowidx])
        pltpu.sync_copy(acc_mn.at[pl.ds(0, RV * CHJ)], mn_out.at[rowidx])
        pltpu.sync_copy(acc_mx.at[pl.ds(0, RV * CHJ)], mx_out.at[rowidx])
        pltpu.sync_copy(degacc, deg_out.at[v])
        return 0

    lax.fori_loop(0, 2 * NJ, step_body, 0)


_edge_kernel = pl.kernel(
    _edge_body,
    out_type=(
        jax.ShapeDtypeStruct((NJ * NV, RV * CHJ), F32),
        jax.ShapeDtypeStruct((NJ * NV, RV * CHJ), F32),
        jax.ShapeDtypeStruct((NJ * NV, RV * CHJ), F32),
        jax.ShapeDtypeStruct((NV, RACC * 16), F32),
    ),
    mesh=_mesh,
    scratch_types=[
        pltpu.VMEM((80,), jnp.int32),
        pltpu.VMEM((C,), jnp.int32),
        pltpu.VMEM((C,), jnp.int32),
        pltpu.VMEM((C,), jnp.int32),
        pltpu.VMEM((C,), jnp.int32),
        pltpu.VMEM((C, CHJ), F32),
        pltpu.VMEM((RACC * CHJ,), F32),
        pltpu.VMEM((RACC * CHJ,), F32),
        pltpu.VMEM((RACC * CHJ,), F32),
        pltpu.VMEM((RACC * CHJ,), F32),
        pltpu.VMEM((RACC * 16,), F32),
        pltpu.SemaphoreType.DMA,
    ],
)


BF16 = jnp.bfloat16


def _dotb(u, w):
    return jnp.dot(u.astype(BF16), w.astype(BF16),
                   preferred_element_type=F32)


def _pre_body(x_ref, wd_ref, ws_ref, b_ref, p_ref, q_ref):
    xb = x_ref[...]
    p_ref[...] = _dotb(xb, wd_ref[...]) + b_ref[...]
    q_ref[...] = _dotb(xb, ws_ref[...])


def _post_body(x_ref, s_ref, mn_ref, mx_ref, deg_ref,
               a_ref, b_ref, c_ref, d_ref, e_ref, f_ref, g_ref,
               pb_ref, idl_ref, lw_ref, lb_ref, o_ref):
    h = x_ref[...]
    deg = deg_ref[...]
    he = deg > 0
    mean = s_ref[...] / jnp.maximum(deg, 1.0)
    mn = jnp.where(he, mn_ref[...], 0.0)
    mx = jnp.where(he, mx_ref[...], 0.0)
    amp = jnp.log(1.0 + deg) * idl_ref[...]
    base = _dotb(h, a_ref[...]) + _dotb(mean, b_ref[...]) \
        + _dotb(mn, c_ref[...]) + _dotb(mx, d_ref[...]) + pb_ref[...]
    scaled = _dotb(mean, e_ref[...]) + _dotb(mn, f_ref[...]) \
        + _dotb(mx, g_ref[...])
    y = base + amp * scaled
    o_ref[...] = jnp.maximum(_dotb(y, lw_ref[...]) + lb_ref[...], 0.0)


def _fc_body(x_ref, w_ref, b_ref, o_ref):
    o_ref[...] = _dotb(x_ref[...], w_ref[...]) + b_ref[...]


def _row_spec(rb):
    return pl.BlockSpec((rb, D), lambda i: (i, 0))


def _full_spec(shape):
    return pl.BlockSpec(shape, lambda i: tuple(0 for _ in shape))


_BR = 2048


def _pre_call(xp, wd, ws, bias):
    return pl.pallas_call(
        _pre_body,
        grid=(NP // _BR,),
        in_specs=[_row_spec(_BR), _full_spec((D, D)), _full_spec((D, D)),
                  _full_spec((1, D))],
        out_specs=[_row_spec(_BR), _row_spec(_BR)],
        out_shape=[jax.ShapeDtypeStruct((NP, D), F32)] * 2,
    )(xp, wd, ws, bias)


def _post_call(xp, s, mn, mx, deg, mats, pb, idl, lw, lb):
    return pl.pallas_call(
        _post_body,
        grid=(NP // _BR,),
        in_specs=[_row_spec(_BR)] * 4
        + [pl.BlockSpec((_BR, 1), lambda i: (i, 0))]
        + [_full_spec((D, D))] * 7
        + [_full_spec((1, D)), _full_spec((1, 1)), _full_spec((D, D)),
           _full_spec((1, D))],
        out_specs=_row_spec(_BR),
        out_shape=jax.ShapeDtypeStruct((NP, D), F32),
    )(xp, s, mn, mx, deg, *mats, pb, idl, lw, lb)


def _fc_call(x, w, b):
    return pl.pallas_call(
        _fc_body,
        grid=(5,),
        in_specs=[pl.BlockSpec((2000, D), lambda i: (i, 0)),
                  _full_spec((D, D)), _full_spec((1, D))],
        out_specs=pl.BlockSpec((2000, D), lambda i: (i, 0)),
        out_shape=jax.ShapeDtypeStruct((N0, D), F32),
    )(x, w, b)


def _blockdiag(w0, w1):
    k = w0.shape[0]
    out = jnp.zeros((2 * k, 2 * w0.shape[1]), F32)
    out = out.at[:k, :w0.shape[1]].set(w0)
    out = out.at[k:, w0.shape[1]:].set(w1)
    return out


def kernel(x, edge_index, pre_W, pre_b, post_W, post_b, lin_W, lin_b, delta, fc_W, fc_b):
    src = edge_index[0]
    dst = edge_index[1]
    tin = D // 2

    xp = jnp.pad(x, ((0, NP - N0), (0, 0)))

    perm = jnp.argsort(dst)
    ds_s = dst[perm]
    sr_s = src[perm]
    bounds = jnp.searchsorted(
        ds_s, jnp.arange(NV + 1, dtype=jnp.int32) * RV).astype(jnp.int32)
    bounds = jnp.pad(bounds, (0, 80 - (NV + 1)), constant_values=E)
    ds_p = jnp.concatenate(
        [ds_s, jnp.full((EP - E,), 1 << 22, jnp.int32)])
    sr_p = jnp.concatenate([sr_s, jnp.zeros((EP - E,), jnp.int32)])

    num_layers = pre_W.shape[0]
    for l in range(num_layers):
        wd = _blockdiag(pre_W[l, 0][:tin], pre_W[l, 1][:tin])
        ws = _blockdiag(pre_W[l, 0][tin:], pre_W[l, 1][tin:])
        bias = jnp.concatenate([pre_b[l, 0], pre_b[l, 1]]).reshape(1, D)

        p2, q2 = _pre_call(xp, wd, ws, bias)
        pstack = p2.reshape(NV, RV, NJ, CHJ).transpose(2, 0, 1, 3) \
                   .reshape(NJ * NV, RV * CHJ)
        qstack = jnp.concatenate(
            [q2[:, j * CHJ:(j + 1) * CHJ] for j in range(NJ)], axis=0)

        s_f, mn_f, mx_f, deg_f = _edge_kernel(
            ds_p, sr_p, bounds, pstack, qstack)
        s = s_f.reshape(NJ, NV, RV, CHJ).transpose(1, 2, 0, 3).reshape(NP, D)
        mn = mn_f.reshape(NJ, NV, RV, CHJ).transpose(1, 2, 0, 3).reshape(NP, D)
        mx = mx_f.reshape(NJ, NV, RV, CHJ).transpose(1, 2, 0, 3).reshape(NP, D)
        deg = deg_f.reshape(NV, RACC, 16)[:, :RV, 0].reshape(NP, 1)

        mats = []
        for r0 in range(0, 7 * tin, tin):
            mats.append(_blockdiag(post_W[l, 0][r0:r0 + tin],
                                   post_W[l, 1][r0:r0 + tin]))
        pb = jnp.concatenate([post_b[l, 0], post_b[l, 1]]).reshape(1, D)
        idl = (1.0 / delta[l]).reshape(1, 1)

        xp = _post_call(xp, s, mn, mx, deg, mats, pb, idl,
                        lin_W[l], lin_b[l].reshape(1, D))

    return _fc_call(xp[:N0], fc_W, fc_b.reshape(1, D))
